# baseline (speedup 1.0000x reference)
import functools

import jax
import jax.numpy as jnp
from jax.experimental import pallas as pl
from jax.experimental.pallas import tpu as pltpu

_E = 8
_NEG_INF = -1e30


def _moe_body(xg_ref, Wg_ref, bg_ref, WeK_ref, be_ref,
              out_ref, aux_ref, w_scr, xprev, WeK_bf, probs_acc, cnt_acc,
              *, blk, n_tokens):
    t = pl.program_id(0)
    nt = pl.num_programs(0)

    @pl.when(t == 0)
    def _cast():
        WeK_bf[...] = WeK_ref[...].astype(jnp.bfloat16)

    @pl.when(t > 0)
    def _experts():
        xb = xprev[...]
        w_dense = w_scr[...]
        xw = jnp.concatenate(
            [(xb * w_dense[:, e:e + 1]).astype(jnp.bfloat16)
             for e in range(_E)],
            axis=1)
        acc = jax.lax.dot_general(
            xw, WeK_bf[...], (((1,), (0,)), ((), ())),
            preferred_element_type=jnp.float32)
        bias = jax.lax.dot_general(
            w_dense, be_ref[...], (((1,), (0,)), ((), ())),
            preferred_element_type=jnp.float32)
        out_ref[...] = acc + bias

    @pl.when(t < nt - 1)
    def _gate():
        xgb = xg_ref[...]
        logits = jax.lax.dot_general(
            xgb, Wg_ref[...], (((1,), (0,)), ((), ())),
            preferred_element_type=jnp.float32) + bg_ref[...]
        iota_e = jax.lax.broadcasted_iota(jnp.int32, (blk, _E), 1)
        max1 = jnp.max(logits, axis=1, keepdims=True)
        idx1 = jnp.min(jnp.where(logits == max1, iota_e, _E), axis=1,
                       keepdims=True)
        masked = jnp.where(iota_e == idx1, _NEG_INF, logits)
        max2 = jnp.max(masked, axis=1, keepdims=True)
        idx2 = jnp.min(jnp.where(masked == max2, iota_e, _E), axis=1,
                       keepdims=True)
        e2 = jnp.exp(max2 - max1)
        w1 = 1.0 / (1.0 + e2)
        w2 = 1.0 - w1
        w_scr[...] = (jnp.where(iota_e == idx1, w1, 0.0) +
                      jnp.where(iota_e == idx2, w2, 0.0))

        probs = jnp.exp(logits - max1)
        probs = probs / jnp.sum(probs, axis=1, keepdims=True)
        block_probs = jnp.sum(probs, axis=0, keepdims=True)
        block_cnt = jnp.sum((iota_e == idx1).astype(jnp.float32), axis=0,
                            keepdims=True)

        @pl.when(t == 0)
        def _init():
            probs_acc[...] = block_probs
            cnt_acc[...] = block_cnt

        @pl.when(t > 0)
        def _accum():
            probs_acc[...] += block_probs
            cnt_acc[...] += block_cnt

        @pl.when(t == nt - 2)
        def _aux():
            aux_ref[...] = jnp.sum(
                cnt_acc[...] / (n_tokens + 1e-8)
                * (probs_acc[...] / n_tokens),
                axis=1, keepdims=True) * _E

        xprev[...] = xgb


def kernel(x, Wg, bg, We, be):
    n, d = x.shape
    blk = 512
    nt = n // blk + 1
    body = functools.partial(_moe_body, blk=blk, n_tokens=n)
    last = n // blk - 1
    out, aux = pl.pallas_call(
        body,
        grid=(nt,),
        in_specs=[
            pl.BlockSpec((blk, d), lambda t: (jnp.minimum(t, last), 0)),
            pl.BlockSpec((d, _E), lambda t: (0, 0)),
            pl.BlockSpec((1, _E), lambda t: (0, 0)),
            pl.BlockSpec((_E * d, d), lambda t: (0, 0)),
            pl.BlockSpec((_E, d), lambda t: (0, 0)),
        ],
        out_specs=[
            pl.BlockSpec((blk, d), lambda t: (jnp.maximum(t - 1, 0), 0)),
            pl.BlockSpec((1, 1), lambda t: (0, 0)),
        ],
        out_shape=[
            jax.ShapeDtypeStruct((n, d), jnp.float32),
            jax.ShapeDtypeStruct((1, 1), jnp.float32),
        ],
        scratch_shapes=[
            pltpu.VMEM((blk, _E), jnp.float32),
            pltpu.VMEM((blk, d), jnp.float32),
            pltpu.VMEM((_E * d, d), jnp.bfloat16),
            pltpu.VMEM((1, _E), jnp.float32),
            pltpu.VMEM((1, _E), jnp.float32),
        ],
        compiler_params=pltpu.CompilerParams(
            dimension_semantics=("arbitrary",)),
    )(x, Wg, bg.reshape(1, _E), We.reshape(_E * d, d), be)
    return out, aux[0, 0]


# speedup vs baseline: 1.0372x; 1.0372x over previous
import functools

import jax
import jax.numpy as jnp
from jax.experimental import pallas as pl
from jax.experimental.pallas import tpu as pltpu

_E = 8
_NEG_INF = -1e30


def _moe_body(xg_ref, Wg_ref, bg_ref, WeK_ref, be_ref,
              out_ref, aux_ref, w_scr, xprev, WeK_bf, probs_acc, cnt_acc,
              *, blk, n_tokens):
    t = pl.program_id(0)
    nt = pl.num_programs(0)

    @pl.when(t == 0)
    def _cast():
        WeK_bf[...] = WeK_ref[...].astype(jnp.bfloat16)

    @pl.when(t > 0)
    def _experts():
        xb = xprev[...]
        w_dense = w_scr[...]
        xw = jnp.concatenate(
            [(xb * w_dense[:, e:e + 1]).astype(jnp.bfloat16)
             for e in range(_E)],
            axis=1)
        acc = jax.lax.dot_general(
            xw, WeK_bf[...], (((1,), (0,)), ((), ())),
            preferred_element_type=jnp.float32)
        bias = jax.lax.dot_general(
            w_dense, be_ref[...], (((1,), (0,)), ((), ())),
            preferred_element_type=jnp.float32)
        out_ref[...] = acc + bias

    @pl.when(t < nt - 1)
    def _gate():
        xgb = xg_ref[...]
        logits = jax.lax.dot_general(
            xgb, Wg_ref[...], (((1,), (0,)), ((), ())),
            preferred_element_type=jnp.float32) + bg_ref[...]
        iota_e = jax.lax.broadcasted_iota(jnp.int32, (blk, _E), 1)
        max1 = jnp.max(logits, axis=1, keepdims=True)
        idx1 = jnp.min(jnp.where(logits == max1, iota_e, _E), axis=1,
                       keepdims=True)
        masked = jnp.where(iota_e == idx1, _NEG_INF, logits)
        max2 = jnp.max(masked, axis=1, keepdims=True)
        idx2 = jnp.min(jnp.where(masked == max2, iota_e, _E), axis=1,
                       keepdims=True)
        e2 = jnp.exp(max2 - max1)
        w1 = 1.0 / (1.0 + e2)
        w2 = 1.0 - w1
        w_scr[...] = (jnp.where(iota_e == idx1, w1, 0.0) +
                      jnp.where(iota_e == idx2, w2, 0.0))

        probs = jnp.exp(logits - max1)
        probs = probs / jnp.sum(probs, axis=1, keepdims=True)
        block_probs = jnp.sum(probs, axis=0, keepdims=True)
        block_cnt = jnp.sum((iota_e == idx1).astype(jnp.float32), axis=0,
                            keepdims=True)

        @pl.when(t == 0)
        def _init():
            probs_acc[...] = block_probs
            cnt_acc[...] = block_cnt

        @pl.when(t > 0)
        def _accum():
            probs_acc[...] += block_probs
            cnt_acc[...] += block_cnt

        @pl.when(t == nt - 2)
        def _aux():
            aux_ref[...] = jnp.sum(
                cnt_acc[...] / (n_tokens + 1e-8)
                * (probs_acc[...] / n_tokens),
                axis=1, keepdims=True) * _E

        xprev[...] = xgb


def kernel(x, Wg, bg, We, be):
    n, d = x.shape
    blk = 1024
    nt = n // blk + 1
    body = functools.partial(_moe_body, blk=blk, n_tokens=n)
    last = n // blk - 1
    out, aux = pl.pallas_call(
        body,
        grid=(nt,),
        in_specs=[
            pl.BlockSpec((blk, d), lambda t: (jnp.minimum(t, last), 0)),
            pl.BlockSpec((d, _E), lambda t: (0, 0)),
            pl.BlockSpec((1, _E), lambda t: (0, 0)),
            pl.BlockSpec((_E * d, d), lambda t: (0, 0)),
            pl.BlockSpec((_E, d), lambda t: (0, 0)),
        ],
        out_specs=[
            pl.BlockSpec((blk, d), lambda t: (jnp.maximum(t - 1, 0), 0)),
            pl.BlockSpec((1, 1), lambda t: (0, 0)),
        ],
        out_shape=[
            jax.ShapeDtypeStruct((n, d), jnp.float32),
            jax.ShapeDtypeStruct((1, 1), jnp.float32),
        ],
        scratch_shapes=[
            pltpu.VMEM((blk, _E), jnp.float32),
            pltpu.VMEM((blk, d), jnp.float32),
            pltpu.VMEM((_E * d, d), jnp.bfloat16),
            pltpu.VMEM((1, _E), jnp.float32),
            pltpu.VMEM((1, _E), jnp.float32),
        ],
        compiler_params=pltpu.CompilerParams(
            dimension_semantics=("arbitrary",)),
    )(x, Wg, bg.reshape(1, _E), We.reshape(_E * d, d), be)
    return out, aux[0, 0]
